# Initial kernel scaffold; baseline (speedup 1.0000x reference)
import jax
import jax.numpy as jnp
from jax.experimental import pallas as pl


def kernel(features, edge_index, edge_weight, W1, b1, W2, b2, Wl, bl):
    raise NotImplementedError("write your pallas kernel here")


# speedup vs baseline: 12.3680x; 12.3680x over previous
import functools

import numpy as np

import jax
import jax.numpy as jnp
from jax import lax
from jax.experimental import pallas as pl
from jax.experimental.pallas import tpu as pltpu
from jax.experimental.pallas import tpu_sc as plsc

N_NODES = 10000
N_EDGES = 320000
D = 128
NC = 2
NS = 16
NW = NC * NS
E_PER_TILE = N_EDGES // NW
CH = 80
NCHUNK = E_PER_TILE // CH
NSUP = 5
G = NCHUNK // NSUP
CR = 80
NCPY = N_NODES // CR
CPT = (NCPY + NS - 1) // NS


def _make_sc_agg():
    mesh = plsc.VectorSubcoreMesh(core_axis_name="c", subcore_axis_name="s")

    @functools.partial(
        pl.kernel,
        mesh=mesh,
        out_type=jax.ShapeDtypeStruct((NC, N_NODES, D), jnp.float32),
        scratch_types=[
            pltpu.VMEM((G, CH), jnp.int32),
            pltpu.VMEM((G, CH), jnp.int32),
            pltpu.VMEM((G, CH), jnp.float32),
            pltpu.VMEM((CH, D), jnp.float32),
            pltpu.VMEM((CH, D), jnp.float32),
            pltpu.VMEM((CH, D), jnp.float32),
            pltpu.VMEM_SHARED((N_NODES, D), jnp.float32),
            pltpu.SemaphoreType.DMA,
            pltpu.SemaphoreType.DMA,
            pltpu.SemaphoreType.DMA,
            pltpu.SemaphoreType.DMA,
            pltpu.SemaphoreType.DMA,
            pltpu.SemaphoreType.DMA,
        ],
    )
    def sc_agg(h_hbm, src_hbm, dst_hbm, w_hbm, out_hbm,
               src_v, dst_v, w_v, rows0_v, rows1_v, rows2_v, acc,
               gsem0, gsem1, gsem2, ssem0, ssem1, ssem2):
        c = lax.axis_index("c")
        s = lax.axis_index("s")
        tile = c * jnp.int32(NS) + s

        bufs = (rows0_v, rows1_v, rows2_v)
        gsems = (gsem0, gsem1, gsem2)
        ssems = (ssem0, ssem1, ssem2)

        zvec = jnp.zeros((16,), jnp.float32)

        def zrow(i, carry):
            for j in range(D // 16):
                rows0_v[i, pl.ds(j * 16, 16)] = zvec
            return carry

        lax.fori_loop(jnp.int32(0), jnp.int32(CR), zrow, jnp.int32(0))
        for t in range(CPT):
            cid = s + jnp.int32(t * NS)

            @pl.when(cid < jnp.int32(NCPY))
            def _():
                r = pl.multiple_of(cid * jnp.int32(CR), CR)
                pltpu.async_copy(rows0_v, acc.at[pl.ds(r, CR)], gsem0)

        for t in range(CPT):
            cid = s + jnp.int32(t * NS)

            @pl.when(cid < jnp.int32(NCPY))
            def _():
                pltpu.make_async_copy(
                    rows0_v, acc.at[pl.ds(jnp.int32(0), CR)], gsem0).wait()

        plsc.subcore_barrier()

        def start_gather(k, b):
            pltpu.make_async_copy(
                h_hbm.at[src_v.at[k]], bufs[b], gsems[b]).start()

        def wait_gather(k, b):
            pltpu.make_async_copy(
                h_hbm.at[src_v.at[k]], bufs[b], gsems[b]).wait()

        def start_scatter(k, b):
            pltpu.async_copy(
                bufs[b], acc.at[dst_v.at[k]], ssems[b], add=True)

        def wait_scatter(k, b):
            pltpu.make_async_copy(
                bufs[b], acc.at[dst_v.at[k]], ssems[b]).wait()

        def scale(k, b):
            buf = bufs[b]

            def grp(g, c2):
                base = g * jnp.int32(16)
                wvec = w_v[k, pl.ds(base, 16)]
                for i in range(16):
                    wb = jnp.full((16,), wvec[i], jnp.float32)
                    row = base + jnp.int32(i)
                    for j in range(D // 16):
                        sl = pl.ds(j * 16, 16)
                        buf[row, sl] = buf[row, sl] * wb
                return c2

            lax.fori_loop(jnp.int32(0), jnp.int32(CH // 16), grp, jnp.int32(0))

        def slot(k, b, nb, first_wait, gather_next):
            wait_gather(k, b)
            scale(k, b)
            start_scatter(k, b)
            if gather_next:
                if first_wait:
                    wait_scatter(k, nb)
                start_gather(k + jnp.int32(2), nb)

        def super_body(u, carry):
            pltpu.sync_copy(src_hbm.at[tile, u], src_v)
            pltpu.sync_copy(dst_hbm.at[tile, u], dst_v)
            pltpu.sync_copy(w_hbm.at[tile, u], w_v)
            start_gather(jnp.int32(0), 0)
            start_gather(jnp.int32(1), 1)
            slot(jnp.int32(0), 0, 2, False, True)

            def triple_body(p, c2):
                k1 = p * jnp.int32(3) + jnp.int32(1)
                slot(k1, 1, 0, True, True)
                slot(k1 + jnp.int32(1), 2, 1, True, True)
                slot(k1 + jnp.int32(2), 0, 2, True, True)
                return c2

            lax.fori_loop(jnp.int32(0), jnp.int32((G - 4) // 3), triple_body,
                          jnp.int32(0))
            slot(jnp.int32(G - 3), 1, 0, True, True)
            slot(jnp.int32(G - 2), 2, 1, False, False)
            slot(jnp.int32(G - 1), 0, 2, False, False)
            wait_scatter(jnp.int32(G - 1), 0)
            wait_scatter(jnp.int32(G - 3), 1)
            wait_scatter(jnp.int32(G - 2), 2)
            return carry

        lax.fori_loop(jnp.int32(0), jnp.int32(NSUP), super_body, jnp.int32(0))
        plsc.subcore_barrier()

        for t in range(CPT):
            cid = s + jnp.int32(t * NS)

            @pl.when(cid < jnp.int32(NCPY))
            def _():
                r = pl.multiple_of(cid * jnp.int32(CR), CR)
                pltpu.sync_copy(acc.at[pl.ds(r, CR)],
                                out_hbm.at[c, pl.ds(r, CR)])

    return sc_agg


_sc_agg = _make_sc_agg()

BM = 1000
_I0 = np.int32(0)


def _tc_mid_body(p0_ref, p1_ref, w1_ref, b1_ref, w2_ref, o_ref):
    agg = p0_ref[...] + p1_ref[...]
    x1 = jnp.tanh(
        jnp.dot(agg, w1_ref[...], preferred_element_type=jnp.float32)
        + b1_ref[...]
    )
    h2 = jnp.dot(x1, w2_ref[...], preferred_element_type=jnp.float32)
    o_ref[...] = jnp.concatenate(
        [h2, jnp.zeros((BM, D - 100), jnp.float32)], axis=1)


def _tc_mid(p0, p1, W1, b1, W2):
    return pl.pallas_call(
        _tc_mid_body,
        grid=(N_NODES // BM,),
        in_specs=[
            pl.BlockSpec((BM, D), lambda i: (i, _I0)),
            pl.BlockSpec((BM, D), lambda i: (i, _I0)),
            pl.BlockSpec((D, 200), lambda i: (_I0, _I0)),
            pl.BlockSpec((1, 200), lambda i: (_I0, _I0)),
            pl.BlockSpec((200, 100), lambda i: (_I0, _I0)),
        ],
        out_specs=pl.BlockSpec((BM, D), lambda i: (i, _I0)),
        out_shape=jax.ShapeDtypeStruct((N_NODES, D), jnp.float32),
    )(p0, p1, W1, b1, W2)


def _tc_fin_body(p0_ref, p1_ref, b2_ref, wl_ref, bl_ref, o_ref):
    agg = p0_ref[...] + p1_ref[...]
    x2 = jnp.tanh(agg[:, :100] + b2_ref[...])
    logits = (
        jnp.dot(x2, wl_ref[...], preferred_element_type=jnp.float32)
        + bl_ref[...]
    )
    mx = jnp.max(logits, axis=-1, keepdims=True)
    e = jnp.exp(logits - mx)
    o_ref[...] = e / jnp.sum(e, axis=-1, keepdims=True)


def _tc_fin(p0, p1, b2, Wl, bl):
    return pl.pallas_call(
        _tc_fin_body,
        grid=(N_NODES // BM,),
        in_specs=[
            pl.BlockSpec((BM, D), lambda i: (i, _I0)),
            pl.BlockSpec((BM, D), lambda i: (i, _I0)),
            pl.BlockSpec((1, 100), lambda i: (_I0, _I0)),
            pl.BlockSpec((100, 10), lambda i: (_I0, _I0)),
            pl.BlockSpec((1, 10), lambda i: (_I0, _I0)),
        ],
        out_specs=pl.BlockSpec((BM, 10), lambda i: (i, _I0)),
        out_shape=jax.ShapeDtypeStruct((N_NODES, 10), jnp.float32),
    )(p0, p1, b2, Wl, bl)


def kernel(features, edge_index, edge_weight, W1, b1, W2, b2, Wl, bl):
    src = edge_index[0].astype(jnp.int32)
    dst = edge_index[1].astype(jnp.int32)
    w = edge_weight.astype(jnp.float32)
    f = features.astype(jnp.float32)

    src3 = src.reshape(NW, NSUP, G, CH)
    dst3 = dst.reshape(NW, NSUP, G, CH)
    w2 = w.reshape(NW, NSUP, G, CH)

    p1 = _sc_agg(f, src3, dst3, w2)
    h2 = _tc_mid(p1[0], p1[1], W1.astype(jnp.float32),
                 b1.astype(jnp.float32)[None, :], W2.astype(jnp.float32))
    p2 = _sc_agg(h2, src3, dst3, w2)
    return _tc_fin(p2[0], p2[1], b2.astype(jnp.float32)[None, :],
                   Wl.astype(jnp.float32), bl.astype(jnp.float32)[None, :])
